# baseline (speedup 1.0000x reference)
import jax
import jax.numpy as jnp
from jax.experimental import pallas as pl

N = 10000
E = 320000
H = 128
G = 256
C = 10
L = 3


def _final_linear_body(r_ref, w_ref, b_ref, o_ref):
    o_ref[...] = r_ref[...] @ w_ref[...] + b_ref[...]


def _bn(x, g, b):
    m = jnp.mean(x, axis=0)
    v = jnp.mean((x - m) ** 2, axis=0)
    return g * (x - m) / jnp.sqrt(v + 1e-5) + b


def _seg_mean(x, ids):
    s = jax.ops.segment_sum(x, ids, num_segments=G)
    c = jax.ops.segment_sum(jnp.ones((x.shape[0], 1), x.dtype), ids, num_segments=G)
    return s / jnp.maximum(c, 1.0)


def _seg_max(x, ids):
    m = jax.ops.segment_max(x, ids, num_segments=G)
    return jnp.where(jnp.isfinite(m), m, 0.0)


def kernel(x_visit, x_service, edge_index_vs, edge_index_sv, batch_visit, batch_service,
           Wrel_vs, brel_vs, Wroot_vs, Wrel_sv, brel_sv, Wroot_sv,
           bn_g_visit, bn_b_visit, bn_g_service, bn_b_service, lin_W, lin_b):
    xv, xs = x_visit, x_service
    readout = jnp.zeros((G, 2 * H), jnp.float32)
    for l in range(L):
        msg_s = jax.ops.segment_sum(xv[edge_index_vs[0]], edge_index_vs[1], num_segments=N)
        out_s = msg_s @ Wrel_vs[l] + brel_vs[l] + xs @ Wroot_vs[l]
        msg_v = jax.ops.segment_sum(xs[edge_index_sv[0]], edge_index_sv[1], num_segments=N)
        out_v = msg_v @ Wrel_sv[l] + brel_sv[l] + xv @ Wroot_sv[l]
        xv = _bn(jax.nn.relu(out_v), bn_g_visit, bn_b_visit)
        xs = _bn(jax.nn.relu(out_s), bn_g_service, bn_b_service)
        mean_pool = _seg_mean(xv, batch_visit) + _seg_mean(xs, batch_service)
        max_pool = _seg_max(xv, batch_visit) + _seg_max(xs, batch_service)
        readout = readout + jnp.concatenate([mean_pool, max_pool], axis=1)
    return pl.pallas_call(
        _final_linear_body,
        out_shape=jax.ShapeDtypeStruct((G, C), jnp.float32),
    )(readout, lin_W, lin_b)


# speedup vs baseline: 2.1128x; 2.1128x over previous
import functools

import jax
import jax.numpy as jnp
from jax import lax
from jax.experimental import pallas as pl
from jax.experimental.pallas import tpu as pltpu
from jax.experimental.pallas import tpu_sc as plsc

N = 10000
E = 320000
H = 128
G = 256
C = 10
L = 3

NC = 2
NS = 16
EDGES_PER_TILE = E // NS
CHUNK = 80
NCHUNKS = EDGES_PER_TILE // CHUNK
NP = 10240
ROWS_PER_TILE = NP // NS


def _seg_sum_sc(x_cat, src_cat, dst_cat, zeros_n):
    mesh = plsc.VectorSubcoreMesh(core_axis_name="c", subcore_axis_name="s")

    @functools.partial(
        pl.kernel,
        out_type=jax.ShapeDtypeStruct((NC, NP, H), jnp.float32),
        mesh=mesh,
        scratch_types=[
            pltpu.VMEM((CHUNK,), jnp.int32),
            pltpu.VMEM((CHUNK,), jnp.int32),
            pltpu.VMEM((CHUNK, H), jnp.float32),
            pltpu.VMEM_SHARED((NP, H), jnp.float32),
            pltpu.SemaphoreType.DMA,
        ],
    )
    def seg_sum_kernel(x_hbm, src_hbm, dst_hbm, zeros_hbm, out_hbm,
                       src_v, dst_v, rows_v, acc_sh, sem):
        c = lax.axis_index("c")
        s = lax.axis_index("s")
        r0 = s * ROWS_PER_TILE
        pltpu.sync_copy(zeros_hbm.at[pl.ds(r0, ROWS_PER_TILE)],
                        acc_sh.at[pl.ds(r0, ROWS_PER_TILE)])
        plsc.subcore_barrier()
        base = c * E + s * EDGES_PER_TILE

        def body(i, carry):
            off = base + i * CHUNK
            pltpu.sync_copy(src_hbm.at[pl.ds(off, CHUNK)], src_v)
            pltpu.sync_copy(dst_hbm.at[pl.ds(off, CHUNK)], dst_v)
            pltpu.async_copy(x_hbm.at[src_v], rows_v, sem).wait()
            pltpu.sync_copy(rows_v, acc_sh.at[dst_v], add=True)
            return carry

        lax.fori_loop(0, NCHUNKS, body, 0)
        plsc.subcore_barrier()
        pltpu.sync_copy(acc_sh.at[pl.ds(r0, ROWS_PER_TILE)],
                        out_hbm.at[c, pl.ds(r0, ROWS_PER_TILE)])

    return seg_sum_kernel(x_cat, src_cat, dst_cat, zeros_n)


def _bn(x, g, b):
    m = jnp.mean(x, axis=0)
    v = jnp.mean((x - m) ** 2, axis=0)
    return g * (x - m) / jnp.sqrt(v + 1e-5) + b


def _seg_mean(x, ids):
    s = jax.ops.segment_sum(x, ids, num_segments=G)
    c = jax.ops.segment_sum(jnp.ones((x.shape[0], 1), x.dtype), ids, num_segments=G)
    return s / jnp.maximum(c, 1.0)


def _seg_max(x, ids):
    m = jax.ops.segment_max(x, ids, num_segments=G)
    return jnp.where(jnp.isfinite(m), m, 0.0)


def _final_linear_body(r_ref, w_ref, b_ref, o_ref):
    o_ref[...] = r_ref[...] @ w_ref[...] + b_ref[...]


def kernel(x_visit, x_service, edge_index_vs, edge_index_sv, batch_visit, batch_service,
           Wrel_vs, brel_vs, Wroot_vs, Wrel_sv, brel_sv, Wroot_sv,
           bn_g_visit, bn_b_visit, bn_g_service, bn_b_service, lin_W, lin_b):
    src_cat = jnp.concatenate([edge_index_vs[0], edge_index_sv[0] + N])
    dst_cat = jnp.concatenate([edge_index_vs[1], edge_index_sv[1]])
    zeros_n = jnp.zeros((NP, H), jnp.float32)
    xv, xs = x_visit, x_service
    readout = jnp.zeros((G, 2 * H), jnp.float32)
    for l in range(L):
        x_cat = jnp.concatenate([xv, xs], axis=0)
        msg = _seg_sum_sc(x_cat, src_cat, dst_cat, zeros_n)
        msg_s, msg_v = msg[0, :N], msg[1, :N]
        out_s = msg_s @ Wrel_vs[l] + brel_vs[l] + xs @ Wroot_vs[l]
        out_v = msg_v @ Wrel_sv[l] + brel_sv[l] + xv @ Wroot_sv[l]
        xv = _bn(jax.nn.relu(out_v), bn_g_visit, bn_b_visit)
        xs = _bn(jax.nn.relu(out_s), bn_g_service, bn_b_service)
        mean_pool = _seg_mean(xv, batch_visit) + _seg_mean(xs, batch_service)
        max_pool = _seg_max(xv, batch_visit) + _seg_max(xs, batch_service)
        readout = readout + jnp.concatenate([mean_pool, max_pool], axis=1)
    return pl.pallas_call(
        _final_linear_body,
        out_shape=jax.ShapeDtypeStruct((G, C), jnp.float32),
    )(readout, lin_W, lin_b)
